# baseline (speedup 1.0000x reference)
import functools

import jax
import jax.numpy as jnp
from jax import lax
from jax.experimental import pallas as pl
from jax.experimental.pallas import tpu as pltpu
from jax.experimental.pallas import tpu_sc as plsc

BATCH = 16384
HIDDEN = 64
FEAT_DIM = 20

_NC = 2
_NS = 16
_NW = _NC * _NS
_BPW = BATCH // _NW
_HB = _BPW // 2


def _sc_gather_body(uti_hbm, mti_hbm, utab_hbm, mtab_hbm,
                    uout_hbm, mout_hbm,
                    uti_v, mti_v, slab, gsem, wsem):
    wid = lax.axis_index("s") * _NC + lax.axis_index("c")
    base = wid * _BPW
    pltpu.sync_copy(uti_hbm.at[pl.ds(base, _BPW)], uti_v)
    pltpu.sync_copy(mti_hbm.at[pl.ds(base, _BPW)], mti_v)

    def gather(idx_ref, tab_hbm, half, par):
        pltpu.make_async_copy(
            tab_hbm.at[idx_ref.at[pl.ds(half * _HB, _HB)]], slab.at[par], gsem
        ).start()

    def wait_gather(tab_hbm, par):
        pltpu.make_async_copy(
            tab_hbm.at[pl.ds(0, _HB)], slab.at[par], gsem).wait()

    def write(out_hbm, half, par):
        pltpu.make_async_copy(
            slab.at[par], out_hbm.at[pl.ds(base + half * _HB, _HB)], wsem
        ).start()

    def wait_write(out_hbm, par):
        pltpu.make_async_copy(
            out_hbm.at[pl.ds(0, _HB)], slab.at[par], wsem).wait()

    gather(uti_v, utab_hbm, 0, 0)
    gather(uti_v, utab_hbm, 1, 1)
    wait_gather(utab_hbm, 0)
    write(uout_hbm, 0, 0)
    wait_gather(utab_hbm, 1)
    write(uout_hbm, 1, 1)
    wait_write(uout_hbm, 0)
    gather(mti_v, mtab_hbm, 0, 0)
    wait_write(uout_hbm, 1)
    gather(mti_v, mtab_hbm, 1, 1)
    wait_gather(mtab_hbm, 0)
    write(mout_hbm, 0, 0)
    wait_gather(mtab_hbm, 1)
    write(mout_hbm, 1, 1)
    wait_write(mout_hbm, 0)
    wait_write(mout_hbm, 1)


@functools.cache
def _sc_gather():
    return pl.kernel(
        _sc_gather_body,
        out_type=(
            jax.ShapeDtypeStruct((BATCH, 2 * HIDDEN), jnp.float32),
            jax.ShapeDtypeStruct((BATCH, 2 * HIDDEN), jnp.float32),
        ),
        mesh=plsc.VectorSubcoreMesh(core_axis_name="c", subcore_axis_name="s"),
        scratch_types=[
            pltpu.VMEM((_BPW,), jnp.int32),
            pltpu.VMEM((_BPW,), jnp.int32),
            pltpu.VMEM((2, _HB, 2 * HIDDEN), jnp.float32),
            pltpu.SemaphoreType.DMA,
            pltpu.SemaphoreType.DMA,
        ],
    )


def _tc_combine_body(feat_ref, us_ref, ms_ref, up_ref, mp_ref, w_ref, b_ref,
                     out_ref):
    proj = jnp.dot(feat_ref[...], w_ref[...],
                   preferred_element_type=jnp.float32) + b_ref[...]
    upar = up_ref[...] == 1
    mpar = mp_ref[...] == 1
    u = jnp.where(upar, us_ref[:, HIDDEN:], us_ref[:, :HIDDEN])
    m = jnp.where(mpar, ms_ref[:, HIDDEN:], ms_ref[:, :HIDDEN])
    out_ref[...] = jnp.sum(u * (m + proj), axis=1).reshape(out_ref.shape)


_TC_ROWS = 2048


def _tc_combine(movie_features, uslab, mslab, upar, mpar, W, b2d):
    grid = (BATCH // _TC_ROWS,)
    out = pl.pallas_call(
        _tc_combine_body,
        grid=grid,
        in_specs=[
            pl.BlockSpec((_TC_ROWS, FEAT_DIM), lambda i: (i, 0)),
            pl.BlockSpec((_TC_ROWS, 2 * HIDDEN), lambda i: (i, 0)),
            pl.BlockSpec((_TC_ROWS, 2 * HIDDEN), lambda i: (i, 0)),
            pl.BlockSpec((_TC_ROWS, 1), lambda i: (i, 0)),
            pl.BlockSpec((_TC_ROWS, 1), lambda i: (i, 0)),
            pl.BlockSpec((FEAT_DIM, HIDDEN), lambda i: (0, 0)),
            pl.BlockSpec((1, HIDDEN), lambda i: (0, 0)),
        ],
        out_specs=pl.BlockSpec((_TC_ROWS,), lambda i: (i,)),
        out_shape=jax.ShapeDtypeStruct((BATCH,), jnp.float32),
    )(movie_features, uslab, mslab, upar, mpar, W, b2d)
    return out


@jax.jit
def kernel(user_ids, movie_ids, movie_features, user_table, movie_table, W, b):
    uids = user_ids.astype(jnp.int32)
    mids = movie_ids.astype(jnp.int32)
    utab2 = user_table.reshape(-1, 2 * HIDDEN)
    mtab2 = movie_table.reshape(-1, 2 * HIDDEN)
    uslab, mslab = _sc_gather()(uids >> 1, mids >> 1, utab2, mtab2)
    return _tc_combine(movie_features, uslab, mslab,
                       (uids & 1).reshape(BATCH, 1),
                       (mids & 1).reshape(BATCH, 1),
                       W, b.reshape(1, HIDDEN))
